# baseline (device time: 19879 ns/iter reference)
import jax
import jax.numpy as jnp
from jax import lax
from jax.experimental import pallas as pl
from jax.experimental.pallas import tpu as pltpu

N_DEV = 4
M = 512
CH = 512
HW = CH // 2
P = 1
SW = HW // P
BF16 = jnp.bfloat16


def kernel(x):
    def body(x_ref, out_ref, sb_r, rb_r, sb_l, rb_l, ss_r, rs_r, ss_l, rs_l):
        my = lax.axis_index("i")
        left = lax.rem(my + N_DEV - 1, N_DEV)
        right = lax.rem(my + 1, N_DEV)

        barrier_sem = pltpu.get_barrier_semaphore()
        for nbr in (left, right):
            pl.semaphore_signal(
                barrier_sem, inc=1,
                device_id=(nbr,), device_id_type=pl.DeviceIdType.MESH,
            )
        pl.semaphore_wait(barrier_sem, 2)

        def mk(dirn, h, s):
            sb, rb, ss, rs = (
                (sb_r, rb_r, ss_r, rs_r) if dirn == 0 else (sb_l, rb_l, ss_l, rs_l)
            )
            tgt = right if dirn == 0 else left
            return pltpu.make_async_remote_copy(
                src_ref=sb.at[h, :, pl.ds(s * SW, SW)],
                dst_ref=rb.at[h, :, pl.ds(s * SW, SW)],
                send_sem=ss.at[h, s],
                recv_sem=rs.at[h, s],
                device_id=(tgt,),
                device_id_type=pl.DeviceIdType.MESH,
            )

        def contrib(dirn, h, s):
            cj = lax.rem(my + 2 + (h if dirn else -h), N_DEV)
            col = cj * CH + (HW if dirn else 0) + s * SW
            return x_ref[0, :, pl.ds(col, SW)].astype(BF16)

        cr0 = lax.rem(my + 3, N_DEV)
        cl0 = lax.rem(my + 1, N_DEV)
        started = {}
        for s in range(P):
            sb_r[0, :, pl.ds(s * SW, SW)] = x_ref[
                0, :, pl.ds(cr0 * CH + s * SW, SW)
            ].astype(BF16)
            started[(0, 0, s)] = mk(0, 0, s)
            started[(0, 0, s)].start()
            sb_l[0, :, pl.ds(s * SW, SW)] = x_ref[
                0, :, pl.ds(cl0 * CH + HW + s * SW, SW)
            ].astype(BF16)
            started[(1, 0, s)] = mk(1, 0, s)
            started[(1, 0, s)].start()

        for h in range(N_DEV - 1):
            for s in range(P):
                for dirn in (0, 1):
                    sb, rb = (sb_r, rb_r) if dirn == 0 else (sb_l, rb_l)
                    started[(dirn, h, s)].wait_recv()
                    acc = rb[h, :, pl.ds(s * SW, SW)] + contrib(dirn, h, s)
                    if h < N_DEV - 2:
                        sb[h + 1, :, pl.ds(s * SW, SW)] = acc
                        started[(dirn, h + 1, s)] = mk(dirn, h + 1, s)
                        started[(dirn, h + 1, s)].start()
                    else:
                        col = (HW if dirn else 0) + s * SW
                        out_ref[:, pl.ds(col, SW)] = acc

        for r in started.values():
            r.wait_send()

    return pl.pallas_call(
        body,
        out_shape=jax.ShapeDtypeStruct((M, CH), BF16),
        in_specs=[pl.BlockSpec(memory_space=pltpu.VMEM)],
        out_specs=pl.BlockSpec(memory_space=pltpu.VMEM),
        scratch_shapes=[
            pltpu.VMEM((N_DEV - 1, M, HW), BF16),
            pltpu.VMEM((N_DEV - 1, M, HW), BF16),
            pltpu.VMEM((N_DEV - 1, M, HW), BF16),
            pltpu.VMEM((N_DEV - 1, M, HW), BF16),
            pltpu.SemaphoreType.DMA((N_DEV - 1, P)),
            pltpu.SemaphoreType.DMA((N_DEV - 1, P)),
            pltpu.SemaphoreType.DMA((N_DEV - 1, P)),
            pltpu.SemaphoreType.DMA((N_DEV - 1, P)),
        ],
        compiler_params=pltpu.CompilerParams(collective_id=0),
    )(x)


# device time: 16673 ns/iter; 1.1923x vs baseline; 1.1923x over previous
import jax
import jax.numpy as jnp
from jax import lax
from jax.experimental import pallas as pl
from jax.experimental.pallas import tpu as pltpu

N_DEV = 4
M = 512
CH = 512
HW = CH // 2
P = 4
MH = M // P
BF16 = jnp.bfloat16


def kernel(x):
    def body(x_ref, out_ref, sb_r, rb_r, sb_l, rb_l, ss_r, rs_r, ss_l, rs_l):
        my = lax.axis_index("i")
        left = lax.rem(my + N_DEV - 1, N_DEV)
        right = lax.rem(my + 1, N_DEV)

        barrier_sem = pltpu.get_barrier_semaphore()
        for nbr in (left, right):
            pl.semaphore_signal(
                barrier_sem, inc=1,
                device_id=(nbr,), device_id_type=pl.DeviceIdType.MESH,
            )
        pl.semaphore_wait(barrier_sem, 2)

        def mk(dirn, h, s):
            sb, rb, ss, rs = (
                (sb_r, rb_r, ss_r, rs_r) if dirn == 0 else (sb_l, rb_l, ss_l, rs_l)
            )
            tgt = right if dirn == 0 else left
            return pltpu.make_async_remote_copy(
                src_ref=sb.at[h, pl.ds(s * MH, MH), :],
                dst_ref=rb.at[h, pl.ds(s * MH, MH), :],
                send_sem=ss.at[h, s],
                recv_sem=rs.at[h, s],
                device_id=(tgt,),
                device_id_type=pl.DeviceIdType.MESH,
            )

        def contrib(dirn, h, s):
            cj = lax.rem(my + 2 + (h if dirn else -h), N_DEV)
            col = cj * CH + (HW if dirn else 0)
            return x_ref[0, pl.ds(s * MH, MH), pl.ds(col, HW)].astype(BF16)

        cr0 = lax.rem(my + 3, N_DEV)
        cl0 = lax.rem(my + 1, N_DEV)
        started = {}
        for s in range(P):
            sb_r[0, pl.ds(s * MH, MH), :] = x_ref[
                0, pl.ds(s * MH, MH), pl.ds(cr0 * CH, HW)
            ].astype(BF16)
            started[(0, 0, s)] = mk(0, 0, s)
            started[(0, 0, s)].start()
            sb_l[0, pl.ds(s * MH, MH), :] = x_ref[
                0, pl.ds(s * MH, MH), pl.ds(cl0 * CH + HW, HW)
            ].astype(BF16)
            started[(1, 0, s)] = mk(1, 0, s)
            started[(1, 0, s)].start()

        for h in range(N_DEV - 1):
            for s in range(P):
                for dirn in (0, 1):
                    sb, rb = (sb_r, rb_r) if dirn == 0 else (sb_l, rb_l)
                    started[(dirn, h, s)].wait_recv()
                    acc = rb[h, pl.ds(s * MH, MH), :] + contrib(dirn, h, s)
                    if h < N_DEV - 2:
                        sb[h + 1, pl.ds(s * MH, MH), :] = acc
                        started[(dirn, h + 1, s)] = mk(dirn, h + 1, s)
                        started[(dirn, h + 1, s)].start()
                    else:
                        col = HW if dirn else 0
                        out_ref[pl.ds(s * MH, MH), pl.ds(col, HW)] = acc

        for r in started.values():
            r.wait_send()

    return pl.pallas_call(
        body,
        out_shape=jax.ShapeDtypeStruct((M, CH), BF16),
        in_specs=[pl.BlockSpec(memory_space=pltpu.VMEM)],
        out_specs=pl.BlockSpec(memory_space=pltpu.VMEM),
        scratch_shapes=[
            pltpu.VMEM((N_DEV - 1, M, HW), BF16),
            pltpu.VMEM((N_DEV - 1, M, HW), BF16),
            pltpu.VMEM((N_DEV - 1, M, HW), BF16),
            pltpu.VMEM((N_DEV - 1, M, HW), BF16),
            pltpu.SemaphoreType.DMA((N_DEV - 1, P)),
            pltpu.SemaphoreType.DMA((N_DEV - 1, P)),
            pltpu.SemaphoreType.DMA((N_DEV - 1, P)),
            pltpu.SemaphoreType.DMA((N_DEV - 1, P)),
        ],
        compiler_params=pltpu.CompilerParams(collective_id=0),
    )(x)


# device time: 15789 ns/iter; 1.2590x vs baseline; 1.0560x over previous
import jax
import jax.numpy as jnp
from jax import lax
from jax.experimental import pallas as pl
from jax.experimental.pallas import tpu as pltpu

N_DEV = 4
M = 512
CH = 512
HW = CH // 2
P = 4
MH = M // P
BF16 = jnp.bfloat16


def kernel(x):
    def body(x_ref, out_ref, sb_r, rb_r, sb_l, rb_l, ss_r, rs_r, ss_l, rs_l):
        my = lax.axis_index("i")
        left = lax.rem(my + N_DEV - 1, N_DEV)
        right = lax.rem(my + 1, N_DEV)

        barrier_sem = pltpu.get_barrier_semaphore()
        for nbr in (left, right):
            pl.semaphore_signal(
                barrier_sem, inc=1,
                device_id=(nbr,), device_id_type=pl.DeviceIdType.MESH,
            )

        def mk(dirn, h, s):
            sb, rb, ss, rs = (
                (sb_r, rb_r, ss_r, rs_r) if dirn == 0 else (sb_l, rb_l, ss_l, rs_l)
            )
            tgt = right if dirn == 0 else left
            return pltpu.make_async_remote_copy(
                src_ref=sb.at[h, pl.ds(s * MH, MH), :],
                dst_ref=rb.at[h, pl.ds(s * MH, MH), :],
                send_sem=ss.at[h, s],
                recv_sem=rs.at[h, s],
                device_id=(tgt,),
                device_id_type=pl.DeviceIdType.MESH,
            )

        def contrib(dirn, h, s):
            cj = lax.rem(my + 2 + (h if dirn else -h), N_DEV)
            col = cj * CH + (HW if dirn else 0)
            return x_ref[0, pl.ds(s * MH, MH), pl.ds(col, HW)].astype(BF16)

        cr0 = lax.rem(my + 3, N_DEV)
        cl0 = lax.rem(my + 1, N_DEV)
        started = {}
        for s in range(P):
            rows = pl.ds(s * MH, MH)
            sb_r[0, rows, :] = x_ref[0, rows, pl.ds(cr0 * CH, HW)].astype(BF16)
            sb_l[0, rows, :] = x_ref[0, rows, pl.ds(cl0 * CH + HW, HW)].astype(BF16)
            if s == 0:
                pl.semaphore_wait(barrier_sem, 2)
            for dirn in (0, 1):
                started[(dirn, 0, s)] = mk(dirn, 0, s)
                started[(dirn, 0, s)].start()

        for h in range(N_DEV - 1):
            for s in range(P):
                for dirn in (0, 1):
                    sb, rb = (sb_r, rb_r) if dirn == 0 else (sb_l, rb_l)
                    started[(dirn, h, s)].wait_recv()
                    rows = pl.ds(s * MH, MH)
                    acc = rb[h, rows, :] + contrib(dirn, h, s)
                    if h < N_DEV - 2:
                        sb[h + 1, rows, :] = acc
                        started[(dirn, h + 1, s)] = mk(dirn, h + 1, s)
                        started[(dirn, h + 1, s)].start()
                    else:
                        col = HW if dirn else 0
                        out_ref[rows, pl.ds(col, HW)] = acc

        for r in started.values():
            r.wait_send()

    return pl.pallas_call(
        body,
        out_shape=jax.ShapeDtypeStruct((M, CH), BF16),
        in_specs=[pl.BlockSpec(memory_space=pltpu.VMEM)],
        out_specs=pl.BlockSpec(memory_space=pltpu.VMEM),
        scratch_shapes=[
            pltpu.VMEM((N_DEV - 1, M, HW), BF16),
            pltpu.VMEM((N_DEV - 1, M, HW), BF16),
            pltpu.VMEM((N_DEV - 1, M, HW), BF16),
            pltpu.VMEM((N_DEV - 1, M, HW), BF16),
            pltpu.SemaphoreType.DMA((N_DEV - 1, P)),
            pltpu.SemaphoreType.DMA((N_DEV - 1, P)),
            pltpu.SemaphoreType.DMA((N_DEV - 1, P)),
            pltpu.SemaphoreType.DMA((N_DEV - 1, P)),
        ],
        compiler_params=pltpu.CompilerParams(collective_id=0),
    )(x)
